# baseline (speedup 1.0000x reference)
import functools

import jax
import jax.numpy as jnp
from jax import lax
from jax.experimental import pallas as pl
from jax.experimental.pallas import tpu as pltpu
import jax.experimental.pallas.tpu_sc as plsc

NC = 2
NS = 16
NW = NC * NS
CH = 128
NCH = 80

_MESH = dict(core_axis_name="c", subcore_axis_name="s", num_cores=NC,
             num_subcores=NS)


def _deg_body(rpt, src_hbm, dst_hbm, onesa_hbm, onesb_hbm, zrow_hbm,
              deg_hbm, deg_sh, onesa_v, onesb_v, sidx_v, didx_v):
    c = lax.axis_index("c")
    s = lax.axis_index("s")
    wid = c * NS + s
    pltpu.sync_copy(onesa_hbm, onesa_v)
    pltpu.sync_copy(onesb_hbm, onesb_v)
    pltpu.sync_copy(zrow_hbm, deg_sh.at[pl.ds(s * rpt, rpt)])
    plsc.subcore_barrier()
    base = wid * NCH * CH

    @pl.loop(0, NCH)
    def _(i):
        off = base + i * CH
        pltpu.sync_copy(src_hbm.at[pl.ds(off, CH)], sidx_v)
        pltpu.sync_copy(dst_hbm.at[pl.ds(off, CH)], didx_v)
        pltpu.sync_copy(onesa_v, deg_sh.at[sidx_v], add=True)
        pltpu.sync_copy(onesb_v, deg_sh.at[didx_v], add=True)

    plsc.subcore_barrier()
    pltpu.sync_copy(deg_sh.at[pl.ds(s * rpt, rpt)],
                    deg_hbm.at[c, pl.ds(s * rpt, rpt)])


def _scatter_body(rpt, h_hbm, src_hbm, dst_hbm, zrow_hbm, out_hbm,
                  agg_sh, rows_v, sidx_v, didx_v):
    c = lax.axis_index("c")
    s = lax.axis_index("s")
    wid = c * NS + s
    pltpu.sync_copy(zrow_hbm, agg_sh.at[pl.ds(s * rpt, rpt)])
    plsc.subcore_barrier()
    base = wid * NCH * CH

    @pl.loop(0, NCH)
    def _(i):
        off = base + i * CH
        pltpu.sync_copy(src_hbm.at[pl.ds(off, CH)], sidx_v)
        pltpu.sync_copy(dst_hbm.at[pl.ds(off, CH)], didx_v)
        pltpu.sync_copy(h_hbm.at[sidx_v], rows_v)
        pltpu.sync_copy(rows_v, agg_sh.at[didx_v], add=True)

    plsc.subcore_barrier()
    pltpu.sync_copy(agg_sh.at[pl.ds(s * rpt, rpt)],
                    out_hbm.at[c, pl.ds(s * rpt, rpt)])


def _make_deg(np_, d):
    rpt = np_ // NS
    body = functools.partial(_deg_body, rpt)
    return pl.kernel(
        body,
        out_type=jax.ShapeDtypeStruct((NC, np_, d), jnp.float32),
        mesh=plsc.VectorSubcoreMesh(**_MESH),
        scratch_types=[
            pltpu.VMEM_SHARED((np_, d), jnp.float32),
            pltpu.VMEM((CH, d), jnp.float32),
            pltpu.VMEM((CH, d), jnp.float32),
            pltpu.VMEM((CH,), jnp.int32),
            pltpu.VMEM((CH,), jnp.int32),
        ],
    )


def _make_scatter(np_, d):
    rpt = np_ // NS
    body = functools.partial(_scatter_body, rpt)
    return pl.kernel(
        body,
        out_type=jax.ShapeDtypeStruct((NC, np_, d), jnp.float32),
        mesh=plsc.VectorSubcoreMesh(**_MESH),
        scratch_types=[
            pltpu.VMEM_SHARED((np_, d), jnp.float32),
            pltpu.VMEM((CH, d), jnp.float32),
            pltpu.VMEM((CH,), jnp.int32),
            pltpu.VMEM((CH,), jnp.int32),
        ],
    )


BR = 1024


def _prep_body(deg_ref, x_ref, soutb_ref, sinb_ref, h0_ref):
    do = deg_ref[0, :, 0:1] + deg_ref[1, :, 0:1]
    di = deg_ref[0, :, 64:65] + deg_ref[1, :, 64:65]
    so = lax.rsqrt(jnp.maximum(do, 1.0))
    si = lax.rsqrt(jnp.maximum(di, 1.0))
    soutb_ref[...] = jnp.broadcast_to(so, soutb_ref.shape)
    sinb_ref[...] = jnp.broadcast_to(si, sinb_ref.shape)
    h0_ref[...] = x_ref[...] * so


def _dense_act_body(p_ref, sinb_ref, soutb_ref, w_ref, b_ref, o_ref):
    agg = (p_ref[0] + p_ref[1]) * sinb_ref[...]
    h = jnp.dot(agg, w_ref[...], preferred_element_type=jnp.float32,
                precision=lax.Precision.HIGHEST) + b_ref[...]
    o_ref[...] = jnp.maximum(h, 0.0) * soutb_ref[...]


def _dense_last_body(p_ref, sinb_ref, w_ref, b_ref, o_ref):
    agg = (p_ref[0] + p_ref[1]) * sinb_ref[...]
    o_ref[...] = jnp.dot(agg, w_ref[...], preferred_element_type=jnp.float32,
                         precision=lax.Precision.HIGHEST) + b_ref[...]


def kernel(x, edge_index, W1, b1, W2, b2, W3, b3):
    n, d = x.shape
    e = edge_index.shape[1]
    np_ = ((n + 2047) // 2048) * 2048
    rpt = np_ // NS
    ep = NW * NCH * CH

    eidx = jnp.concatenate(
        [edge_index.astype(jnp.int32),
         jnp.full((2, ep - e), n, jnp.int32)], axis=1)
    src1 = eidx[0]
    dst1 = eidx[1]
    x_p = jnp.pad(x, ((0, np_ - n), (0, 0)))

    lane = jnp.arange(d, dtype=jnp.int32)
    onesa = jnp.broadcast_to((lane == 0).astype(jnp.float32), (CH, d))
    onesb = jnp.broadcast_to((lane == 64).astype(jnp.float32), (CH, d))
    zrow = jnp.zeros((rpt, d), jnp.float32)

    deg_fn = _make_deg(np_, d)
    scat_fn = _make_scatter(np_, d)

    degp = deg_fn(src1, dst1, onesa, onesb, zrow)

    grid = (np_ // BR,)
    soutb, sinb, h0 = pl.pallas_call(
        _prep_body,
        grid=grid,
        in_specs=[
            pl.BlockSpec((NC, BR, d), lambda i: (0, i, 0)),
            pl.BlockSpec((BR, d), lambda i: (i, 0)),
        ],
        out_specs=[
            pl.BlockSpec((BR, d), lambda i: (i, 0)),
            pl.BlockSpec((BR, d), lambda i: (i, 0)),
            pl.BlockSpec((BR, d), lambda i: (i, 0)),
        ],
        out_shape=[jax.ShapeDtypeStruct((np_, d), jnp.float32)] * 3,
    )(degp, x_p)

    dense_act = pl.pallas_call(
        _dense_act_body,
        grid=grid,
        in_specs=[
            pl.BlockSpec((NC, BR, d), lambda i: (0, i, 0)),
            pl.BlockSpec((BR, d), lambda i: (i, 0)),
            pl.BlockSpec((BR, d), lambda i: (i, 0)),
            pl.BlockSpec((d, d), lambda i: (0, 0)),
            pl.BlockSpec((1, d), lambda i: (0, 0)),
        ],
        out_specs=pl.BlockSpec((BR, d), lambda i: (i, 0)),
        out_shape=jax.ShapeDtypeStruct((np_, d), jnp.float32),
    )
    dense_last = pl.pallas_call(
        _dense_last_body,
        grid=grid,
        in_specs=[
            pl.BlockSpec((NC, BR, d), lambda i: (0, i, 0)),
            pl.BlockSpec((BR, d), lambda i: (i, 0)),
            pl.BlockSpec((d, d), lambda i: (0, 0)),
            pl.BlockSpec((1, d), lambda i: (0, 0)),
        ],
        out_specs=pl.BlockSpec((BR, d), lambda i: (i, 0)),
        out_shape=jax.ShapeDtypeStruct((np_, d), jnp.float32),
    )

    b1r = b1.reshape(1, d)
    b2r = b2.reshape(1, d)
    b3r = b3.reshape(1, d)

    p = scat_fn(h0, src1, dst1, zrow)
    h1 = dense_act(p, sinb, soutb, W1, b1r)
    p = scat_fn(h1, src1, dst1, zrow)
    h2 = dense_act(p, sinb, soutb, W2, b2r)
    p = scat_fn(h2, src1, dst1, zrow)
    h3 = dense_last(p, sinb, W3, b3r)
    return h3[:n]


# speedup vs baseline: 1.9035x; 1.9035x over previous
import functools

import jax
import jax.numpy as jnp
from jax import lax
from jax.experimental import pallas as pl
from jax.experimental.pallas import tpu as pltpu
import jax.experimental.pallas.tpu_sc as plsc

NC = 2
NS = 16
NW = NC * NS
CH = 128
NCH = 80

_MESH = dict(core_axis_name="c", subcore_axis_name="s", num_cores=NC,
             num_subcores=NS)


def _deg_body(rpt, src_hbm, dst_hbm, onesa_hbm, onesb_hbm, zrow_hbm,
              deg_hbm, deg_sh, onesa_v, onesb_v, sidx_v, didx_v):
    c = lax.axis_index("c")
    s = lax.axis_index("s")
    wid = c * NS + s
    pltpu.sync_copy(onesa_hbm, onesa_v)
    pltpu.sync_copy(onesb_hbm, onesb_v)
    pltpu.sync_copy(zrow_hbm, deg_sh.at[pl.ds(s * rpt, rpt)])
    plsc.subcore_barrier()
    base = wid * NCH * CH

    @pl.loop(0, NCH)
    def _(i):
        off = base + i * CH
        pltpu.sync_copy(src_hbm.at[pl.ds(off, CH)], sidx_v)
        pltpu.sync_copy(dst_hbm.at[pl.ds(off, CH)], didx_v)
        pltpu.sync_copy(onesa_v, deg_sh.at[sidx_v], add=True)
        pltpu.sync_copy(onesb_v, deg_sh.at[didx_v], add=True)

    plsc.subcore_barrier()
    pltpu.sync_copy(deg_sh.at[pl.ds(s * rpt, rpt)],
                    deg_hbm.at[c, pl.ds(s * rpt, rpt)])


def _scatter_body(rpt, h_hbm, src_hbm, dst_hbm, zrow_hbm, out_hbm,
                  agg_sh, rows_v, sidx_v, didx_v):
    c = lax.axis_index("c")
    s = lax.axis_index("s")
    wid = c * NS + s
    pltpu.sync_copy(zrow_hbm, agg_sh.at[pl.ds(s * rpt, rpt)])
    plsc.subcore_barrier()
    base = wid * NCH * CH

    @pl.loop(0, NCH)
    def _(i):
        off = base + i * CH
        pltpu.sync_copy(src_hbm.at[pl.ds(off, CH)], sidx_v)
        pltpu.sync_copy(dst_hbm.at[pl.ds(off, CH)], didx_v)
        pltpu.sync_copy(h_hbm.at[sidx_v], rows_v)
        pltpu.sync_copy(rows_v, agg_sh.at[didx_v], add=True)

    plsc.subcore_barrier()
    pltpu.sync_copy(agg_sh.at[pl.ds(s * rpt, rpt)],
                    out_hbm.at[c, pl.ds(s * rpt, rpt)])


def _make_deg(np_, d):
    rpt = np_ // NS
    body = functools.partial(_deg_body, rpt)
    return pl.kernel(
        body,
        out_type=jax.ShapeDtypeStruct((NC, np_, d), jnp.float32),
        mesh=plsc.VectorSubcoreMesh(**_MESH),
        scratch_types=[
            pltpu.VMEM_SHARED((np_, d), jnp.float32),
            pltpu.VMEM((CH, d), jnp.float32),
            pltpu.VMEM((CH, d), jnp.float32),
            pltpu.VMEM((CH,), jnp.int32),
            pltpu.VMEM((CH,), jnp.int32),
        ],
    )


def _make_scatter(np_, d):
    rpt = np_ // NS
    body = functools.partial(_scatter_body, rpt)
    return pl.kernel(
        body,
        out_type=jax.ShapeDtypeStruct((NC, np_, d), jnp.float32),
        mesh=plsc.VectorSubcoreMesh(**_MESH),
        scratch_types=[
            pltpu.VMEM_SHARED((np_, d), jnp.float32),
            pltpu.VMEM((CH, d), jnp.float32),
            pltpu.VMEM((CH,), jnp.int32),
            pltpu.VMEM((CH,), jnp.int32),
        ],
    )


BR = 1024


def _prep_body(deg_ref, x_ref, soutb_ref, sinb_ref, h0_ref):
    do = deg_ref[0, :, 0:1] + deg_ref[1, :, 0:1]
    di = deg_ref[0, :, 64:65] + deg_ref[1, :, 64:65]
    so = lax.rsqrt(jnp.maximum(do, 1.0))
    si = lax.rsqrt(jnp.maximum(di, 1.0))
    soutb_ref[...] = jnp.broadcast_to(so, soutb_ref.shape)
    sinb_ref[...] = jnp.broadcast_to(si, sinb_ref.shape)
    h0_ref[...] = x_ref[...] * so


def _dense_act_body(p_ref, sinb_ref, soutb_ref, w_ref, b_ref, o_ref):
    agg = (p_ref[0] + p_ref[1]) * sinb_ref[...]
    h = jnp.dot(agg, w_ref[...], preferred_element_type=jnp.float32,
                precision=lax.Precision.HIGHEST) + b_ref[...]
    o_ref[...] = jnp.maximum(h, 0.0) * soutb_ref[...]


def _dense_last_body(p_ref, sinb_ref, w_ref, b_ref, o_ref):
    agg = (p_ref[0] + p_ref[1]) * sinb_ref[...]
    o_ref[...] = jnp.dot(agg, w_ref[...], preferred_element_type=jnp.float32,
                         precision=lax.Precision.HIGHEST) + b_ref[...]


def kernel(x, edge_index, W1, b1, W2, b2, W3, b3):
    n, d = x.shape
    e = edge_index.shape[1]
    np_ = ((n + 2047) // 2048) * 2048
    rpt = np_ // NS
    ep = NW * NCH * CH

    pad_row = n + (jnp.arange(ep - e, dtype=jnp.int32) % (np_ - n))
    eidx = jnp.concatenate(
        [edge_index.astype(jnp.int32),
         jnp.stack([pad_row, pad_row])], axis=1)
    src1 = eidx[0]
    dst1 = eidx[1]
    x_p = jnp.pad(x, ((0, np_ - n), (0, 0)))

    lane = jnp.arange(d, dtype=jnp.int32)
    onesa = jnp.broadcast_to((lane == 0).astype(jnp.float32), (CH, d))
    onesb = jnp.broadcast_to((lane == 64).astype(jnp.float32), (CH, d))
    zrow = jnp.zeros((rpt, d), jnp.float32)

    deg_fn = _make_deg(np_, d)
    scat_fn = _make_scatter(np_, d)

    degp = deg_fn(src1, dst1, onesa, onesb, zrow)

    grid = (np_ // BR,)
    soutb, sinb, h0 = pl.pallas_call(
        _prep_body,
        grid=grid,
        in_specs=[
            pl.BlockSpec((NC, BR, d), lambda i: (0, i, 0)),
            pl.BlockSpec((BR, d), lambda i: (i, 0)),
        ],
        out_specs=[
            pl.BlockSpec((BR, d), lambda i: (i, 0)),
            pl.BlockSpec((BR, d), lambda i: (i, 0)),
            pl.BlockSpec((BR, d), lambda i: (i, 0)),
        ],
        out_shape=[jax.ShapeDtypeStruct((np_, d), jnp.float32)] * 3,
    )(degp, x_p)

    dense_act = pl.pallas_call(
        _dense_act_body,
        grid=grid,
        in_specs=[
            pl.BlockSpec((NC, BR, d), lambda i: (0, i, 0)),
            pl.BlockSpec((BR, d), lambda i: (i, 0)),
            pl.BlockSpec((BR, d), lambda i: (i, 0)),
            pl.BlockSpec((d, d), lambda i: (0, 0)),
            pl.BlockSpec((1, d), lambda i: (0, 0)),
        ],
        out_specs=pl.BlockSpec((BR, d), lambda i: (i, 0)),
        out_shape=jax.ShapeDtypeStruct((np_, d), jnp.float32),
    )
    dense_last = pl.pallas_call(
        _dense_last_body,
        grid=grid,
        in_specs=[
            pl.BlockSpec((NC, BR, d), lambda i: (0, i, 0)),
            pl.BlockSpec((BR, d), lambda i: (i, 0)),
            pl.BlockSpec((d, d), lambda i: (0, 0)),
            pl.BlockSpec((1, d), lambda i: (0, 0)),
        ],
        out_specs=pl.BlockSpec((BR, d), lambda i: (i, 0)),
        out_shape=jax.ShapeDtypeStruct((np_, d), jnp.float32),
    )

    b1r = b1.reshape(1, d)
    b2r = b2.reshape(1, d)
    b3r = b3.reshape(1, d)

    p = scat_fn(h0, src1, dst1, zrow)
    h1 = dense_act(p, sinb, soutb, W1, b1r)
    p = scat_fn(h1, src1, dst1, zrow)
    h2 = dense_act(p, sinb, soutb, W2, b2r)
    p = scat_fn(h2, src1, dst1, zrow)
    h3 = dense_last(p, sinb, W3, b3r)
    return h3[:n]
